# baseline (speedup 1.0000x reference)
import functools

import numpy as np
import jax
import jax.numpy as jnp
from jax import lax
from jax.experimental import pallas as pl
from jax.experimental.pallas import tpu as pltpu
from jax.experimental.pallas import tpu_sc as plsc

N_WORDS_ = 8192
TOP_K_ = 50
EMB_ = 64
KPAD_ = 64

NC_ = 2
NS_ = 16
NW_ = NC_ * NS_
LANES_ = 16

ROWS_W_ = N_WORDS_ // NW_
CROWS_ = 16
NCHUNK_ = ROWS_W_ // CROWS_

_THRESHOLDS = (2, 3, 4, 5, 8, 16, 32, 64)

_GDN = lax.GatherDimensionNumbers(
    offset_dims=(), collapsed_slice_dims=(0,), start_index_map=(0,))


def _vgather(tv, b):
  return lax.gather(tv, b[:, None], dimension_numbers=_GDN,
                    slice_sizes=(1,),
                    mode=lax.GatherScatterMode.PROMISE_IN_BOUNDS)


def _body(ti_hbm, embt_hbm, out_hbm,
          idx_a, idx_b, bkt_v, rows_a, rows_b, table_v,
          isem_a, isem_b, osem_a, osem_b):
  wid = lax.axis_index("s") * NC_ + lax.axis_index("c")
  i0w = wid * ROWS_W_
  iota = lax.iota(jnp.int32, LANES_)

  pltpu.sync_copy(embt_hbm, table_v)

  def start_idx(c, idx_v, isem):
    ci = i0w + (c % NCHUNK_) * CROWS_
    pltpu.async_copy(ti_hbm.at[:, pl.ds(ci, CROWS_)], idx_v, isem)

  def process(j, c, idx_v, rows_v, isem, osem):
    ci = i0w + c * CROWS_
    it = lax.shift_right_logical(ci, 7)
    ii = pl.multiple_of(lax.bitwise_and(ci, 127), CROWS_)
    wv = ci + iota
    pltpu.make_async_copy(ti_hbm.at[:, pl.ds(ci, CROWS_)],
                          idx_v, isem).wait()

    def bkt_body(k, carry):
      t = idx_v[k, :]
      d = jnp.maximum(wv - t, 1)
      b = jnp.where(d >= 2, 1, 0)
      for thr in _THRESHOLDS[1:]:
        b = b + jnp.where(d >= thr, 1, 0)
      bkt_v[k, :] = b
      return carry

    lax.fori_loop(0, TOP_K_, bkt_body, 0)
    start_idx(c + 2, idx_v, isem)

    @pl.when(j > 0)
    def _():
      pltpu.make_async_copy(
          rows_v, out_hbm.at[:, :, it, :, pl.ds(ii, CROWS_)], osem).wait()

    @plsc.parallel_loop(0, EMB_ // 8, unroll=1)
    def _(et):
      e0 = et * 8
      tvs = [table_v[e0 + q, pl.ds(0, LANES_)] for q in range(8)]
      for k in range(TOP_K_):
        bk = bkt_v[k, :]
        for q in range(8):
          rows_v[k, et, q, :] = _vgather(tvs[q], bk)

    pltpu.async_copy(rows_v, out_hbm.at[:, :, it, :, pl.ds(ii, CROWS_)],
                     osem)

  start_idx(0, idx_a, isem_a)
  start_idx(1, idx_b, isem_b)

  def chunk_pair(j, carry):
    process(j, 2 * j, idx_a, rows_a, isem_a, osem_a)
    process(j, 2 * j + 1, idx_b, rows_b, isem_b, osem_b)
    return carry

  lax.fori_loop(0, NCHUNK_ // 2, chunk_pair, 0)

  it0 = lax.shift_right_logical(i0w, 7)
  ii0 = pl.multiple_of(lax.bitwise_and(i0w, 127), CROWS_)
  pltpu.make_async_copy(rows_a,
                        out_hbm.at[:, :, it0, :, pl.ds(ii0, CROWS_)],
                        osem_a).wait()
  pltpu.make_async_copy(rows_b,
                        out_hbm.at[:, :, it0, :, pl.ds(ii0, CROWS_)],
                        osem_b).wait()
  pltpu.make_async_copy(ti_hbm.at[:, pl.ds(i0w, CROWS_)],
                        idx_a, isem_a).wait()
  pltpu.make_async_copy(ti_hbm.at[:, pl.ds(i0w, CROWS_)],
                        idx_b, isem_b).wait()


@jax.jit
def kernel(top_indices, distance_emb):
  ti_t = jnp.pad(top_indices.astype(jnp.int32),
                 ((0, 0), (0, KPAD_ - TOP_K_))).T
  emb_t = jnp.pad(distance_emb.T, ((0, 0), (0, 128 - 9)))
  run = pl.kernel(
      _body,
      out_type=jax.ShapeDtypeStruct((TOP_K_, EMB_ // 8, N_WORDS_ // 128,
                                     8, 128), jnp.float32),
      mesh=plsc.VectorSubcoreMesh(core_axis_name="c", subcore_axis_name="s"),
      scratch_types=[
          pltpu.VMEM((KPAD_, CROWS_), jnp.int32),
          pltpu.VMEM((KPAD_, CROWS_), jnp.int32),
          pltpu.VMEM((TOP_K_, CROWS_), jnp.int32),
          pltpu.VMEM((TOP_K_, EMB_ // 8, 8, CROWS_), jnp.float32),
          pltpu.VMEM((TOP_K_, EMB_ // 8, 8, CROWS_), jnp.float32),
          pltpu.VMEM((EMB_, 128), jnp.float32),
          pltpu.SemaphoreType.DMA,
          pltpu.SemaphoreType.DMA,
          pltpu.SemaphoreType.DMA,
          pltpu.SemaphoreType.DMA,
      ],
      compiler_params=pltpu.CompilerParams(use_tc_tiling_on_sc=False),
  )
  out = run(ti_t, emb_t)
  return out.transpose(2, 4, 0, 1, 3).reshape(N_WORDS_, TOP_K_, EMB_)


# speedup vs baseline: 1.1240x; 1.1240x over previous
import functools

import numpy as np
import jax
import jax.numpy as jnp
from jax import lax
from jax.experimental import pallas as pl
from jax.experimental.pallas import tpu as pltpu
from jax.experimental.pallas import tpu_sc as plsc

N_WORDS_ = 8192
TOP_K_ = 50
EMB_ = 64
KPAD_ = 64

NC_ = 2
NS_ = 16
NW_ = NC_ * NS_
LANES_ = 16

ROWS_W_ = N_WORDS_ // NW_
CROWS_ = 16
NCHUNK_ = ROWS_W_ // CROWS_

_THRESHOLDS = (2, 3, 4, 5, 8, 16, 32, 64)

_GDN = lax.GatherDimensionNumbers(
    offset_dims=(), collapsed_slice_dims=(0,), start_index_map=(0,))


def _vgather(tv, b):
  return lax.gather(tv, b[:, None], dimension_numbers=_GDN,
                    slice_sizes=(1,),
                    mode=lax.GatherScatterMode.PROMISE_IN_BOUNDS)


def _body(ti_hbm, embt_hbm, out_hbm,
          idx_a, idx_b, bkt_v, rows_a, rows_b, table_v,
          isem_a, isem_b, osem_a, osem_b):
  wid = lax.axis_index("s") * NC_ + lax.axis_index("c")
  i0w = wid * ROWS_W_
  iota = lax.iota(jnp.int32, LANES_)

  pltpu.sync_copy(embt_hbm, table_v)

  def start_idx(c, idx_v, isem):
    ci = i0w + (c % NCHUNK_) * CROWS_
    pltpu.async_copy(ti_hbm.at[:, pl.ds(ci, CROWS_)], idx_v, isem)

  def process(j, c, idx_v, rows_v, isem, osem):
    ci = i0w + c * CROWS_
    it = lax.shift_right_logical(ci, 7)
    ii = pl.multiple_of(lax.bitwise_and(ci, 127), CROWS_)
    wv = ci + iota
    pltpu.make_async_copy(ti_hbm.at[:, pl.ds(ci, CROWS_)],
                          idx_v, isem).wait()

    def bkt_body(k, carry):
      t = idx_v[k, :]
      d = jnp.maximum(wv - t, 1)
      b = jnp.where(d >= 2, 1, 0)
      for thr in _THRESHOLDS[1:]:
        b = b + jnp.where(d >= thr, 1, 0)
      bkt_v[k, :] = b
      return carry

    lax.fori_loop(0, TOP_K_, bkt_body, 0)
    start_idx(c + 2, idx_v, isem)

    @pl.when(j > 0)
    def _():
      pltpu.make_async_copy(
          rows_v, out_hbm.at[:, :, it, :, pl.ds(ii, CROWS_)], osem).wait()

    @plsc.parallel_loop(0, EMB_, unroll=4)
    def _(e):
      tv = table_v[e, pl.ds(0, LANES_)]
      et = lax.shift_right_logical(e, 3)
      ei = lax.bitwise_and(e, 7)
      for k in range(TOP_K_):
        rows_v[k, et, ei, :] = _vgather(tv, bkt_v[k, :])

    pltpu.async_copy(rows_v, out_hbm.at[:, :, it, :, pl.ds(ii, CROWS_)],
                     osem)

  start_idx(0, idx_a, isem_a)
  start_idx(1, idx_b, isem_b)

  def chunk_pair(j, carry):
    process(j, 2 * j, idx_a, rows_a, isem_a, osem_a)
    process(j, 2 * j + 1, idx_b, rows_b, isem_b, osem_b)
    return carry

  lax.fori_loop(0, NCHUNK_ // 2, chunk_pair, 0)

  it0 = lax.shift_right_logical(i0w, 7)
  ii0 = pl.multiple_of(lax.bitwise_and(i0w, 127), CROWS_)
  pltpu.make_async_copy(rows_a,
                        out_hbm.at[:, :, it0, :, pl.ds(ii0, CROWS_)],
                        osem_a).wait()
  pltpu.make_async_copy(rows_b,
                        out_hbm.at[:, :, it0, :, pl.ds(ii0, CROWS_)],
                        osem_b).wait()
  pltpu.make_async_copy(ti_hbm.at[:, pl.ds(i0w, CROWS_)],
                        idx_a, isem_a).wait()
  pltpu.make_async_copy(ti_hbm.at[:, pl.ds(i0w, CROWS_)],
                        idx_b, isem_b).wait()


@jax.jit
def kernel(top_indices, distance_emb):
  ti_t = jnp.pad(top_indices.astype(jnp.int32),
                 ((0, 0), (0, KPAD_ - TOP_K_))).T
  emb_t = jnp.pad(distance_emb.T, ((0, 0), (0, 128 - 9)))
  run = pl.kernel(
      _body,
      out_type=jax.ShapeDtypeStruct((TOP_K_, EMB_ // 8, N_WORDS_ // 128,
                                     8, 128), jnp.float32),
      mesh=plsc.VectorSubcoreMesh(core_axis_name="c", subcore_axis_name="s"),
      scratch_types=[
          pltpu.VMEM((KPAD_, CROWS_), jnp.int32),
          pltpu.VMEM((KPAD_, CROWS_), jnp.int32),
          pltpu.VMEM((TOP_K_, CROWS_), jnp.int32),
          pltpu.VMEM((TOP_K_, EMB_ // 8, 8, CROWS_), jnp.float32),
          pltpu.VMEM((TOP_K_, EMB_ // 8, 8, CROWS_), jnp.float32),
          pltpu.VMEM((EMB_, 128), jnp.float32),
          pltpu.SemaphoreType.DMA,
          pltpu.SemaphoreType.DMA,
          pltpu.SemaphoreType.DMA,
          pltpu.SemaphoreType.DMA,
      ],
      compiler_params=pltpu.CompilerParams(use_tc_tiling_on_sc=False),
  )
  out = run(ti_t, emb_t)
  return out.transpose(2, 4, 0, 1, 3).reshape(N_WORDS_, TOP_K_, EMB_)
